# baseline (speedup 1.0000x reference)
import functools

import jax
import jax.numpy as jnp
from jax import lax
from jax.experimental import pallas as pl
from jax.experimental.pallas import tpu as pltpu
from jax.experimental.pallas import tpu_sc as plsc

N = 10000
E = 160000
IN_DIM, HID, OUT_DIM, NE = 256, 512, 256, 8

NROWS = 10240
RPT = NROWS // 16
EB = 128
EP = 163840
G_SPMM = EP // (16 * EB)
G_DEG = EP // (32 * EB)
RB = 1000
GRID = N // RB

DOT = functools.partial(jnp.dot, preferred_element_type=jnp.float32)

@functools.cache
def _make_deg_sc():
    mesh = plsc.VectorSubcoreMesh(core_axis_name="c", subcore_axis_name="s")

    @functools.partial(
        pl.kernel, mesh=mesh,
        out_type=jax.ShapeDtypeStruct((2, NROWS, 128), jnp.float32),
        scratch_types=[
            pltpu.VMEM((EB,), jnp.int32),
            pltpu.VMEM((EB, 128), jnp.float32),
            pltpu.VMEM((EB, 128), jnp.float32),
            pltpu.VMEM_SHARED((NROWS, 128), jnp.float32),
        ],
    )
    def deg_sc(dst_hbm, ones_hbm, z128_hbm, out_hbm, didx, ones, stage, acc):
        c = lax.axis_index("c")
        s = lax.axis_index("s")
        wid = s * 2 + c
        row0 = s * RPT
        pltpu.sync_copy(ones_hbm, ones)
        pltpu.sync_copy(z128_hbm, stage)
        for k in range(RPT // EB):
            pltpu.sync_copy(stage, acc.at[pl.ds(row0 + k * EB, EB)])
        plsc.subcore_barrier()

        def body(g, carry):
            pltpu.sync_copy(dst_hbm.at[pl.ds(wid * (G_DEG * EB) + g * EB, EB)],
                            didx)
            pltpu.sync_copy(ones, acc.at[didx], add=True)
            return carry

        lax.fori_loop(0, G_DEG, body, 0)
        plsc.subcore_barrier()
        for k in range(RPT // EB):
            pltpu.sync_copy(acc.at[pl.ds(row0 + k * EB, EB)], stage)
            pltpu.sync_copy(stage, out_hbm.at[c, pl.ds(row0 + k * EB, EB)])

    return deg_sc


NB = 5
GB = 64
GPT = EP // (16 * GB)
CH = 16
NCHUNK = GPT // CH


@functools.cache
def _make_segsum(ncb):
    bpc = ncb // 2
    mesh = plsc.VectorSubcoreMesh(core_axis_name="c", subcore_axis_name="s")

    @functools.partial(
        pl.kernel, mesh=mesh,
        out_type=jax.ShapeDtypeStruct((ncb, NROWS, 128), jnp.float32),
        scratch_types=[
            pltpu.VMEM((CH, GB), jnp.int32),
            pltpu.VMEM((CH, GB), jnp.int32),
            pltpu.VMEM((NB, GB, 128), jnp.float32),
            pltpu.VMEM_SHARED((NROWS, 128), jnp.float32),
            pltpu.SemaphoreType.DMA,
            pltpu.SemaphoreType.DMA,
        ],
    )
    def segsum(y_hbm, srcoff_hbm, dst2_hbm, zrows_hbm, out_hbm,
               sidx, didx, rows, acc, gsem, ssem):
        c = lax.axis_index("c")
        s = lax.axis_index("s")
        row0 = s * RPT
        t0 = s * GPT
        for j in range(bpc):
            cb = c * bpc + j
            pltpu.sync_copy(zrows_hbm, acc.at[pl.ds(row0, RPT)])
            plsc.subcore_barrier()

            def body(i, carry):
                g0 = t0 + i * CH
                pltpu.sync_copy(srcoff_hbm.at[cb, pl.ds(g0, CH)], sidx)
                pltpu.sync_copy(dst2_hbm.at[pl.ds(g0, CH)], didx)
                gh = {k: pltpu.async_copy(y_hbm.at[sidx.at[k]],
                                          rows.at[k % NB], gsem)
                      for k in range(NB - 2)}
                sh = {}
                for k in range(CH):
                    if k >= 2:
                        sh[k - 2].wait()
                    kk = k + NB - 2
                    if kk < CH:
                        gh[kk] = pltpu.async_copy(y_hbm.at[sidx.at[kk]],
                                                  rows.at[kk % NB], gsem)
                    gh[k].wait()
                    sh[k] = pltpu.async_copy(rows.at[k % NB],
                                             acc.at[didx.at[k]],
                                             ssem, add=True)
                sh[CH - 2].wait()
                sh[CH - 1].wait()
                return carry

            lax.fori_loop(0, NCHUNK, body, 0)
            plsc.subcore_barrier()
            pltpu.sync_copy(acc.at[pl.ds(row0, RPT)],
                            out_hbm.at[cb, pl.ds(row0, RPT)])
            plsc.subcore_barrier()

    return segsum


def _srcoff_body(s_ref, o_ref):
    v = s_ref[...]
    for cb in range(4):
        o_ref[cb] = v + cb * N


_srcoff = pl.pallas_call(
    _srcoff_body, grid=(1,),
    in_specs=[pl.BlockSpec((EP // EB, EB), lambda i: (0, 0))],
    out_specs=pl.BlockSpec((4, EP // EB, EB), lambda i: (0, 0, 0)),
    out_shape=jax.ShapeDtypeStruct((4, EP // EB, EB), jnp.int32),
)


def _dinv(deg_ref):
    deg = deg_ref[0, :, 0] + deg_ref[1, :, 0]
    return jnp.where(deg > 0, lax.rsqrt(deg), 0.0)[:, None]


def _tc1_body(x_ref, w1_ref, deg_ref, y_ref):
    y = DOT(x_ref[...], w1_ref[...]) * _dinv(deg_ref)
    for cb in range(4):
        y_ref[cb] = y[:, cb * 128:(cb + 1) * 128]


def _tc2_body(s1_ref, deg_ref, b1_ref, wg_ref, y2_ref, gates_ref):
    dinv = _dinv(deg_ref)
    hs = []
    for cb in range(4):
        h_cb = jnp.maximum(
            s1_ref[cb] * dinv + b1_ref[0, cb * 128:(cb + 1) * 128][None, :], 0.0)
        y2_ref[cb] = h_cb * dinv
        hs.append(h_cb)
    h = jnp.concatenate(hs, axis=1)
    logits = DOT(h, wg_ref[...])
    tri = (lax.broadcasted_iota(jnp.int32, (NE, NE), 0)
           <= lax.broadcasted_iota(jnp.int32, (NE, NE), 1)).astype(jnp.float32)
    m1 = jnp.max(logits, axis=1, keepdims=True)
    is1 = (logits == m1).astype(jnp.float32)
    fm1 = jnp.where(jnp.logical_and(is1 > 0, DOT(is1, tri) < 1.5), 1.0, 0.0)
    masked = jnp.where(fm1 > 0, -1e30, logits)
    m2 = jnp.max(masked, axis=1, keepdims=True)
    is2 = (masked == m2).astype(jnp.float32)
    fm2 = jnp.where(jnp.logical_and(is2 > 0, DOT(is2, tri) < 1.5), 1.0, 0.0)
    g1 = 1.0 / (1.0 + jnp.exp(m2 - m1))
    gates_ref[...] = fm1 * g1 + fm2 * (1.0 - g1)


def _tc3_body(s2_ref, deg_ref, gates_ref, we_ref, be_ref, w2_ref, y3_ref):
    dinv = _dinv(deg_ref)
    ax = jnp.concatenate([s2_ref[cb] * dinv for cb in range(4)], axis=1)
    g = gates_ref[...]
    acc = DOT(g, be_ref[...])
    for e in range(NE):
        acc = acc + g[:, e:e + 1] * DOT(ax, we_ref[e])
    y3 = DOT(jnp.maximum(acc, 0.0), w2_ref[...]) * dinv
    for cb in range(4):
        y3_ref[cb] = y3[:, cb * 128:(cb + 1) * 128]


def _tc4_body(s3_ref, deg_ref, b2_ref, w3_ref, y4_ref):
    dinv = _dinv(deg_ref)
    h = jnp.concatenate(
        [jnp.maximum(s3_ref[cb] * dinv
                     + b2_ref[0, cb * 128:(cb + 1) * 128][None, :], 0.0)
         for cb in range(4)], axis=1)
    y4 = DOT(h, w3_ref[...]) * dinv
    for cb in range(2):
        y4_ref[cb] = y4[:, cb * 128:(cb + 1) * 128]


def _tc5_body(s4_ref, deg_ref, b3_ref, out_ref):
    dinv = _dinv(deg_ref)
    o = jnp.concatenate([s4_ref[cb] * dinv for cb in range(2)], axis=1)
    out_ref[...] = o + b3_ref[...]


def _deg_spec():
    return pl.BlockSpec((2, RB, 128), lambda i: (0, i, 0))


def _blk_spec(ncb):
    return pl.BlockSpec((ncb, RB, 128), lambda i: (0, i, 0))


def _full(shape):
    nd = len(shape)
    return pl.BlockSpec(shape, lambda i, _nd=nd: (0,) * _nd)


_tc1 = pl.pallas_call(
    _tc1_body, grid=(GRID,),
    in_specs=[pl.BlockSpec((RB, IN_DIM), lambda i: (i, 0)),
              _full((IN_DIM, HID)), _deg_spec()],
    out_specs=_blk_spec(4),
    out_shape=jax.ShapeDtypeStruct((4, N, 128), jnp.float32),
)

_tc2 = pl.pallas_call(
    _tc2_body, grid=(GRID,),
    in_specs=[_blk_spec(4), _deg_spec(), _full((1, HID)), _full((HID, NE))],
    out_specs=[_blk_spec(4), pl.BlockSpec((RB, NE), lambda i: (i, 0))],
    out_shape=[jax.ShapeDtypeStruct((4, N, 128), jnp.float32),
               jax.ShapeDtypeStruct((N, NE), jnp.float32)],
)

_tc3 = pl.pallas_call(
    _tc3_body, grid=(GRID,),
    in_specs=[_blk_spec(4), _deg_spec(),
              pl.BlockSpec((RB, NE), lambda i: (i, 0)),
              _full((NE, HID, HID)), _full((NE, HID)), _full((HID, HID))],
    out_specs=_blk_spec(4),
    out_shape=jax.ShapeDtypeStruct((4, N, 128), jnp.float32),
)

_tc4 = pl.pallas_call(
    _tc4_body, grid=(GRID,),
    in_specs=[_blk_spec(4), _deg_spec(), _full((1, HID)),
              _full((HID, OUT_DIM))],
    out_specs=_blk_spec(2),
    out_shape=jax.ShapeDtypeStruct((2, N, 128), jnp.float32),
)

_tc5 = pl.pallas_call(
    _tc5_body, grid=(GRID,),
    in_specs=[_blk_spec(2), _deg_spec(), _full((1, OUT_DIM))],
    out_specs=pl.BlockSpec((RB, OUT_DIM), lambda i: (i, 0)),
    out_shape=jax.ShapeDtypeStruct((N, OUT_DIM), jnp.float32),
)


def kernel(x, edge_index, W1, b1, w_gate, We, be, W2, b2, W3, b3):
    dst = edge_index[0]
    src = edge_index[1]
    pad = EP - E
    srcp = jnp.concatenate([src, jnp.zeros((pad,), jnp.int32)])
    dstp = jnp.concatenate([dst, jnp.full((pad,), N, jnp.int32)])
    ones128 = jnp.ones((EB, 128), jnp.float32)
    z128 = jnp.zeros((EB, 128), jnp.float32)
    zrows = jnp.zeros((RPT, 128), jnp.float32)
    srcoff = _srcoff(srcp.reshape(EP // EB, EB)).reshape(4, EP // GB, GB)
    dst2 = dstp.reshape(EP // GB, GB)

    _deg = _make_deg_sc()
    _segsum4 = _make_segsum(4)
    _segsum2 = _make_segsum(2)

    degraw = _deg(dstp, ones128, z128)

    y1 = _tc1(x, W1, degraw)
    s1 = _segsum4(y1.reshape(4 * N, 128), srcoff, dst2, zrows)
    y2, gates = _tc2(s1[:, :N], degraw, b1.reshape(1, -1), w_gate)
    s2 = _segsum4(y2.reshape(4 * N, 128), srcoff, dst2, zrows)
    y3 = _tc3(s2[:, :N], degraw, gates, We, be, W2)
    s3 = _segsum4(y3.reshape(4 * N, 128), srcoff, dst2, zrows)
    y4 = _tc4(s3[:, :N], degraw, b2.reshape(1, -1), W3)
    s4 = _segsum2(y4.reshape(2 * N, 128), srcoff, dst2, zrows)
    out = _tc5(s4[:, :N], degraw, b3.reshape(1, -1))
    return out


# speedup vs baseline: 1.0023x; 1.0023x over previous
import functools

import jax
import jax.numpy as jnp
from jax import lax
from jax.experimental import pallas as pl
from jax.experimental.pallas import tpu as pltpu
from jax.experimental.pallas import tpu_sc as plsc

N = 10000
E = 160000
IN_DIM, HID, OUT_DIM, NE = 256, 512, 256, 8

NROWS = 10240
RPT = NROWS // 16
EB = 128
EP = 163840
G_SPMM = EP // (16 * EB)
G_DEG = EP // (32 * EB)
RB = 1000
GRID = N // RB

DOT = functools.partial(jnp.dot, preferred_element_type=jnp.float32)

@functools.cache
def _make_deg_sc():
    mesh = plsc.VectorSubcoreMesh(core_axis_name="c", subcore_axis_name="s")

    @functools.partial(
        pl.kernel, mesh=mesh,
        out_type=jax.ShapeDtypeStruct((2, NROWS, 128), jnp.float32),
        scratch_types=[
            pltpu.VMEM((EB,), jnp.int32),
            pltpu.VMEM((EB, 128), jnp.float32),
            pltpu.VMEM((EB, 128), jnp.float32),
            pltpu.VMEM_SHARED((NROWS, 128), jnp.float32),
        ],
    )
    def deg_sc(dst_hbm, ones_hbm, z128_hbm, out_hbm, didx, ones, stage, acc):
        c = lax.axis_index("c")
        s = lax.axis_index("s")
        wid = s * 2 + c
        row0 = s * RPT
        pltpu.sync_copy(ones_hbm, ones)
        pltpu.sync_copy(z128_hbm, stage)
        for k in range(RPT // EB):
            pltpu.sync_copy(stage, acc.at[pl.ds(row0 + k * EB, EB)])
        plsc.subcore_barrier()

        def body(g, carry):
            pltpu.sync_copy(dst_hbm.at[pl.ds(wid * (G_DEG * EB) + g * EB, EB)],
                            didx)
            pltpu.sync_copy(ones, acc.at[didx], add=True)
            return carry

        lax.fori_loop(0, G_DEG, body, 0)
        plsc.subcore_barrier()
        for k in range(RPT // EB):
            pltpu.sync_copy(acc.at[pl.ds(row0 + k * EB, EB)], stage)
            pltpu.sync_copy(stage, out_hbm.at[c, pl.ds(row0 + k * EB, EB)])

    return deg_sc


NB = 4
GB = 64
GPT = EP // (16 * GB)
CH = 16
NCHUNK = GPT // CH


@functools.cache
def _make_segsum(ncb):
    bpc = ncb // 2
    mesh = plsc.VectorSubcoreMesh(core_axis_name="c", subcore_axis_name="s")

    @functools.partial(
        pl.kernel, mesh=mesh,
        out_type=jax.ShapeDtypeStruct((ncb, NROWS, 128), jnp.float32),
        scratch_types=[
            pltpu.VMEM((CH, GB), jnp.int32),
            pltpu.VMEM((CH, GB), jnp.int32),
            pltpu.VMEM((NB, GB, 128), jnp.float32),
            pltpu.VMEM_SHARED((NROWS, 128), jnp.float32),
            pltpu.SemaphoreType.DMA,
            pltpu.SemaphoreType.DMA,
        ],
    )
    def segsum(y_hbm, srcoff_hbm, dst2_hbm, zrows_hbm, out_hbm,
               sidx, didx, rows, acc, gsem, ssem):
        c = lax.axis_index("c")
        s = lax.axis_index("s")
        row0 = s * RPT
        t0 = s * GPT
        for j in range(bpc):
            cb = c * bpc + j
            pltpu.sync_copy(zrows_hbm, acc.at[pl.ds(row0, RPT)])
            plsc.subcore_barrier()

            def body(i, carry):
                g0 = t0 + i * CH
                pltpu.sync_copy(srcoff_hbm.at[cb, pl.ds(g0, CH)], sidx)
                pltpu.sync_copy(dst2_hbm.at[pl.ds(g0, CH)], didx)
                gh = {k: pltpu.async_copy(y_hbm.at[sidx.at[k]],
                                          rows.at[k % NB], gsem)
                      for k in range(NB - 2)}
                sh = {}
                for k in range(CH):
                    if k >= 2:
                        sh[k - 2].wait()
                    kk = k + NB - 2
                    if kk < CH:
                        gh[kk] = pltpu.async_copy(y_hbm.at[sidx.at[kk]],
                                                  rows.at[kk % NB], gsem)
                    gh[k].wait()
                    sh[k] = pltpu.async_copy(rows.at[k % NB],
                                             acc.at[didx.at[k]],
                                             ssem, add=True)
                sh[CH - 2].wait()
                sh[CH - 1].wait()
                return carry

            lax.fori_loop(0, NCHUNK, body, 0)
            plsc.subcore_barrier()
            pltpu.sync_copy(acc.at[pl.ds(row0, RPT)],
                            out_hbm.at[cb, pl.ds(row0, RPT)])
            plsc.subcore_barrier()

    return segsum


def _srcoff_body(s_ref, o_ref):
    v = s_ref[...]
    for cb in range(4):
        o_ref[cb] = v + cb * N


_srcoff = pl.pallas_call(
    _srcoff_body, grid=(1,),
    in_specs=[pl.BlockSpec((EP // EB, EB), lambda i: (0, 0))],
    out_specs=pl.BlockSpec((4, EP // EB, EB), lambda i: (0, 0, 0)),
    out_shape=jax.ShapeDtypeStruct((4, EP // EB, EB), jnp.int32),
)


def _dinv(deg_ref):
    deg = deg_ref[0, :, 0] + deg_ref[1, :, 0]
    return jnp.where(deg > 0, lax.rsqrt(deg), 0.0)[:, None]


def _tc1_body(x_ref, w1_ref, deg_ref, y_ref):
    y = DOT(x_ref[...], w1_ref[...]) * _dinv(deg_ref)
    for cb in range(4):
        y_ref[cb] = y[:, cb * 128:(cb + 1) * 128]


def _tc2_body(s1_ref, deg_ref, b1_ref, wg_ref, y2_ref, gates_ref):
    dinv = _dinv(deg_ref)
    hs = []
    for cb in range(4):
        h_cb = jnp.maximum(
            s1_ref[cb] * dinv + b1_ref[0, cb * 128:(cb + 1) * 128][None, :], 0.0)
        y2_ref[cb] = h_cb * dinv
        hs.append(h_cb)
    h = jnp.concatenate(hs, axis=1)
    logits = DOT(h, wg_ref[...])
    tri = (lax.broadcasted_iota(jnp.int32, (NE, NE), 0)
           <= lax.broadcasted_iota(jnp.int32, (NE, NE), 1)).astype(jnp.float32)
    m1 = jnp.max(logits, axis=1, keepdims=True)
    is1 = (logits == m1).astype(jnp.float32)
    fm1 = jnp.where(jnp.logical_and(is1 > 0, DOT(is1, tri) < 1.5), 1.0, 0.0)
    masked = jnp.where(fm1 > 0, -1e30, logits)
    m2 = jnp.max(masked, axis=1, keepdims=True)
    is2 = (masked == m2).astype(jnp.float32)
    fm2 = jnp.where(jnp.logical_and(is2 > 0, DOT(is2, tri) < 1.5), 1.0, 0.0)
    g1 = 1.0 / (1.0 + jnp.exp(m2 - m1))
    gates_ref[...] = fm1 * g1 + fm2 * (1.0 - g1)


def _tc3_body(s2_ref, deg_ref, gates_ref, we_ref, be_ref, w2_ref, y3_ref):
    dinv = _dinv(deg_ref)
    ax = jnp.concatenate([s2_ref[cb] * dinv for cb in range(4)], axis=1)
    g = gates_ref[...]
    acc = DOT(g, be_ref[...])
    for e in range(NE):
        acc = acc + g[:, e:e + 1] * DOT(ax, we_ref[e])
    y3 = DOT(jnp.maximum(acc, 0.0), w2_ref[...]) * dinv
    for cb in range(4):
        y3_ref[cb] = y3[:, cb * 128:(cb + 1) * 128]


def _tc4_body(s3_ref, deg_ref, b2_ref, w3_ref, y4_ref):
    dinv = _dinv(deg_ref)
    h = jnp.concatenate(
        [jnp.maximum(s3_ref[cb] * dinv
                     + b2_ref[0, cb * 128:(cb + 1) * 128][None, :], 0.0)
         for cb in range(4)], axis=1)
    y4 = DOT(h, w3_ref[...]) * dinv
    for cb in range(2):
        y4_ref[cb] = y4[:, cb * 128:(cb + 1) * 128]


def _tc5_body(s4_ref, deg_ref, b3_ref, out_ref):
    dinv = _dinv(deg_ref)
    o = jnp.concatenate([s4_ref[cb] * dinv for cb in range(2)], axis=1)
    out_ref[...] = o + b3_ref[...]


def _deg_spec():
    return pl.BlockSpec((2, RB, 128), lambda i: (0, i, 0))


def _blk_spec(ncb):
    return pl.BlockSpec((ncb, RB, 128), lambda i: (0, i, 0))


def _full(shape):
    nd = len(shape)
    return pl.BlockSpec(shape, lambda i, _nd=nd: (0,) * _nd)


_tc1 = pl.pallas_call(
    _tc1_body, grid=(GRID,),
    in_specs=[pl.BlockSpec((RB, IN_DIM), lambda i: (i, 0)),
              _full((IN_DIM, HID)), _deg_spec()],
    out_specs=_blk_spec(4),
    out_shape=jax.ShapeDtypeStruct((4, N, 128), jnp.float32),
)

_tc2 = pl.pallas_call(
    _tc2_body, grid=(GRID,),
    in_specs=[_blk_spec(4), _deg_spec(), _full((1, HID)), _full((HID, NE))],
    out_specs=[_blk_spec(4), pl.BlockSpec((RB, NE), lambda i: (i, 0))],
    out_shape=[jax.ShapeDtypeStruct((4, N, 128), jnp.float32),
               jax.ShapeDtypeStruct((N, NE), jnp.float32)],
)

_tc3 = pl.pallas_call(
    _tc3_body, grid=(GRID,),
    in_specs=[_blk_spec(4), _deg_spec(),
              pl.BlockSpec((RB, NE), lambda i: (i, 0)),
              _full((NE, HID, HID)), _full((NE, HID)), _full((HID, HID))],
    out_specs=_blk_spec(4),
    out_shape=jax.ShapeDtypeStruct((4, N, 128), jnp.float32),
)

_tc4 = pl.pallas_call(
    _tc4_body, grid=(GRID,),
    in_specs=[_blk_spec(4), _deg_spec(), _full((1, HID)),
              _full((HID, OUT_DIM))],
    out_specs=_blk_spec(2),
    out_shape=jax.ShapeDtypeStruct((2, N, 128), jnp.float32),
)

_tc5 = pl.pallas_call(
    _tc5_body, grid=(GRID,),
    in_specs=[_blk_spec(2), _deg_spec(), _full((1, OUT_DIM))],
    out_specs=pl.BlockSpec((RB, OUT_DIM), lambda i: (i, 0)),
    out_shape=jax.ShapeDtypeStruct((N, OUT_DIM), jnp.float32),
)


def kernel(x, edge_index, W1, b1, w_gate, We, be, W2, b2, W3, b3):
    dst = edge_index[0]
    src = edge_index[1]
    pad = EP - E
    srcp = jnp.concatenate([src, jnp.zeros((pad,), jnp.int32)])
    dstp = jnp.concatenate([dst, jnp.full((pad,), N, jnp.int32)])
    ones128 = jnp.ones((EB, 128), jnp.float32)
    z128 = jnp.zeros((EB, 128), jnp.float32)
    zrows = jnp.zeros((RPT, 128), jnp.float32)
    srcoff = _srcoff(srcp.reshape(EP // EB, EB)).reshape(4, EP // GB, GB)
    dst2 = dstp.reshape(EP // GB, GB)

    _deg = _make_deg_sc()
    _segsum4 = _make_segsum(4)
    _segsum2 = _make_segsum(2)

    degraw = _deg(dstp, ones128, z128)

    y1 = _tc1(x, W1, degraw)
    s1 = _segsum4(y1.reshape(4 * N, 128), srcoff, dst2, zrows)
    y2, gates = _tc2(s1[:, :N], degraw, b1.reshape(1, -1), w_gate)
    s2 = _segsum4(y2.reshape(4 * N, 128), srcoff, dst2, zrows)
    y3 = _tc3(s2[:, :N], degraw, gates, We, be, W2)
    s3 = _segsum4(y3.reshape(4 * N, 128), srcoff, dst2, zrows)
    y4 = _tc4(s3[:, :N], degraw, b2.reshape(1, -1), W3)
    s4 = _segsum2(y4.reshape(2 * N, 128), srcoff, dst2, zrows)
    out = _tc5(s4[:, :N], degraw, b3.reshape(1, -1))
    return out


# speedup vs baseline: 1.0249x; 1.0225x over previous
import functools

import jax
import jax.numpy as jnp
from jax import lax
from jax.experimental import pallas as pl
from jax.experimental.pallas import tpu as pltpu
from jax.experimental.pallas import tpu_sc as plsc

N = 10000
E = 160000
IN_DIM, HID, OUT_DIM, NE = 256, 512, 256, 8

NROWS = 10240
RPT = NROWS // 16
EB = 128
EP = 163840
G_SPMM = EP // (16 * EB)
G_DEG = EP // (32 * EB)
RB = 1000
GRID = N // RB

DOT = functools.partial(jnp.dot, preferred_element_type=jnp.float32)

@functools.cache
def _make_deg_sc():
    mesh = plsc.VectorSubcoreMesh(core_axis_name="c", subcore_axis_name="s")

    @functools.partial(
        pl.kernel, mesh=mesh,
        out_type=jax.ShapeDtypeStruct((2, NROWS, 128), jnp.float32),
        scratch_types=[
            pltpu.VMEM((EB,), jnp.int32),
            pltpu.VMEM((EB, 128), jnp.float32),
            pltpu.VMEM((EB, 128), jnp.float32),
            pltpu.VMEM_SHARED((NROWS, 128), jnp.float32),
        ],
    )
    def deg_sc(dst_hbm, ones_hbm, z128_hbm, out_hbm, didx, ones, stage, acc):
        c = lax.axis_index("c")
        s = lax.axis_index("s")
        wid = s * 2 + c
        row0 = s * RPT
        pltpu.sync_copy(ones_hbm, ones)
        pltpu.sync_copy(z128_hbm, stage)
        for k in range(RPT // EB):
            pltpu.sync_copy(stage, acc.at[pl.ds(row0 + k * EB, EB)])
        plsc.subcore_barrier()

        def body(g, carry):
            pltpu.sync_copy(dst_hbm.at[pl.ds(wid * (G_DEG * EB) + g * EB, EB)],
                            didx)
            pltpu.sync_copy(ones, acc.at[didx], add=True)
            return carry

        lax.fori_loop(0, G_DEG, body, 0)
        plsc.subcore_barrier()
        for k in range(RPT // EB):
            pltpu.sync_copy(acc.at[pl.ds(row0 + k * EB, EB)], stage)
            pltpu.sync_copy(stage, out_hbm.at[c, pl.ds(row0 + k * EB, EB)])

    return deg_sc


NB = 4
GB = 64
GPT = EP // (16 * GB)
CH = 32
NCHUNK = GPT // CH


@functools.cache
def _make_segsum(ncb):
    bpc = ncb // 2
    mesh = plsc.VectorSubcoreMesh(core_axis_name="c", subcore_axis_name="s")

    @functools.partial(
        pl.kernel, mesh=mesh,
        out_type=jax.ShapeDtypeStruct((ncb, NROWS, 128), jnp.float32),
        scratch_types=[
            pltpu.VMEM((CH, GB), jnp.int32),
            pltpu.VMEM((CH, GB), jnp.int32),
            pltpu.VMEM((NB, GB, 128), jnp.float32),
            pltpu.VMEM_SHARED((NROWS, 128), jnp.float32),
            pltpu.SemaphoreType.DMA,
            pltpu.SemaphoreType.DMA,
        ],
    )
    def segsum(y_hbm, srcoff_hbm, dst2_hbm, zrows_hbm, out_hbm,
               sidx, didx, rows, acc, gsem, ssem):
        c = lax.axis_index("c")
        s = lax.axis_index("s")
        row0 = s * RPT
        t0 = s * GPT
        for j in range(bpc):
            cb = c * bpc + j
            pltpu.sync_copy(zrows_hbm, acc.at[pl.ds(row0, RPT)])
            plsc.subcore_barrier()

            def body(i, carry):
                g0 = t0 + i * CH
                pltpu.sync_copy(srcoff_hbm.at[cb, pl.ds(g0, CH)], sidx)
                pltpu.sync_copy(dst2_hbm.at[pl.ds(g0, CH)], didx)
                gh = {k: pltpu.async_copy(y_hbm.at[sidx.at[k]],
                                          rows.at[k % NB], gsem)
                      for k in range(NB - 2)}
                sh = {}
                for k in range(CH):
                    if k >= 2:
                        sh[k - 2].wait()
                    kk = k + NB - 2
                    if kk < CH:
                        gh[kk] = pltpu.async_copy(y_hbm.at[sidx.at[kk]],
                                                  rows.at[kk % NB], gsem)
                    gh[k].wait()
                    sh[k] = pltpu.async_copy(rows.at[k % NB],
                                             acc.at[didx.at[k]],
                                             ssem, add=True)
                sh[CH - 2].wait()
                sh[CH - 1].wait()
                return carry

            lax.fori_loop(0, NCHUNK, body, 0)
            plsc.subcore_barrier()
            pltpu.sync_copy(acc.at[pl.ds(row0, RPT)],
                            out_hbm.at[cb, pl.ds(row0, RPT)])
            plsc.subcore_barrier()

    return segsum


def _srcoff_body(s_ref, o_ref):
    v = s_ref[...]
    for cb in range(4):
        o_ref[cb] = v + cb * N


_srcoff = pl.pallas_call(
    _srcoff_body, grid=(1,),
    in_specs=[pl.BlockSpec((EP // EB, EB), lambda i: (0, 0))],
    out_specs=pl.BlockSpec((4, EP // EB, EB), lambda i: (0, 0, 0)),
    out_shape=jax.ShapeDtypeStruct((4, EP // EB, EB), jnp.int32),
)


def _dinv(deg_ref):
    deg = deg_ref[0, :, 0] + deg_ref[1, :, 0]
    return jnp.where(deg > 0, lax.rsqrt(deg), 0.0)[:, None]


def _tc1_body(x_ref, w1_ref, deg_ref, y_ref):
    y = DOT(x_ref[...], w1_ref[...]) * _dinv(deg_ref)
    for cb in range(4):
        y_ref[cb] = y[:, cb * 128:(cb + 1) * 128]


def _tc2_body(s1_ref, deg_ref, b1_ref, wg_ref, y2_ref, gates_ref):
    dinv = _dinv(deg_ref)
    hs = []
    for cb in range(4):
        h_cb = jnp.maximum(
            s1_ref[cb] * dinv + b1_ref[0, cb * 128:(cb + 1) * 128][None, :], 0.0)
        y2_ref[cb] = h_cb * dinv
        hs.append(h_cb)
    h = jnp.concatenate(hs, axis=1)
    logits = DOT(h, wg_ref[...])
    tri = (lax.broadcasted_iota(jnp.int32, (NE, NE), 0)
           <= lax.broadcasted_iota(jnp.int32, (NE, NE), 1)).astype(jnp.float32)
    m1 = jnp.max(logits, axis=1, keepdims=True)
    is1 = (logits == m1).astype(jnp.float32)
    fm1 = jnp.where(jnp.logical_and(is1 > 0, DOT(is1, tri) < 1.5), 1.0, 0.0)
    masked = jnp.where(fm1 > 0, -1e30, logits)
    m2 = jnp.max(masked, axis=1, keepdims=True)
    is2 = (masked == m2).astype(jnp.float32)
    fm2 = jnp.where(jnp.logical_and(is2 > 0, DOT(is2, tri) < 1.5), 1.0, 0.0)
    g1 = 1.0 / (1.0 + jnp.exp(m2 - m1))
    gates_ref[...] = fm1 * g1 + fm2 * (1.0 - g1)


def _tc3_body(s2_ref, deg_ref, gates_ref, we_ref, be_ref, w2_ref, y3_ref):
    dinv = _dinv(deg_ref)
    ax = jnp.concatenate([s2_ref[cb] * dinv for cb in range(4)], axis=1)
    g = gates_ref[...]
    acc = DOT(g, be_ref[...])
    for e in range(NE):
        acc = acc + g[:, e:e + 1] * DOT(ax, we_ref[e])
    y3 = DOT(jnp.maximum(acc, 0.0), w2_ref[...]) * dinv
    for cb in range(4):
        y3_ref[cb] = y3[:, cb * 128:(cb + 1) * 128]


def _tc4_body(s3_ref, deg_ref, b2_ref, w3_ref, y4_ref):
    dinv = _dinv(deg_ref)
    h = jnp.concatenate(
        [jnp.maximum(s3_ref[cb] * dinv
                     + b2_ref[0, cb * 128:(cb + 1) * 128][None, :], 0.0)
         for cb in range(4)], axis=1)
    y4 = DOT(h, w3_ref[...]) * dinv
    for cb in range(2):
        y4_ref[cb] = y4[:, cb * 128:(cb + 1) * 128]


def _tc5_body(s4_ref, deg_ref, b3_ref, out_ref):
    dinv = _dinv(deg_ref)
    o = jnp.concatenate([s4_ref[cb] * dinv for cb in range(2)], axis=1)
    out_ref[...] = o + b3_ref[...]


def _deg_spec():
    return pl.BlockSpec((2, RB, 128), lambda i: (0, i, 0))


def _blk_spec(ncb):
    return pl.BlockSpec((ncb, RB, 128), lambda i: (0, i, 0))


def _full(shape):
    nd = len(shape)
    return pl.BlockSpec(shape, lambda i, _nd=nd: (0,) * _nd)


_tc1 = pl.pallas_call(
    _tc1_body, grid=(GRID,),
    in_specs=[pl.BlockSpec((RB, IN_DIM), lambda i: (i, 0)),
              _full((IN_DIM, HID)), _deg_spec()],
    out_specs=_blk_spec(4),
    out_shape=jax.ShapeDtypeStruct((4, N, 128), jnp.float32),
)

_tc2 = pl.pallas_call(
    _tc2_body, grid=(GRID,),
    in_specs=[_blk_spec(4), _deg_spec(), _full((1, HID)), _full((HID, NE))],
    out_specs=[_blk_spec(4), pl.BlockSpec((RB, NE), lambda i: (i, 0))],
    out_shape=[jax.ShapeDtypeStruct((4, N, 128), jnp.float32),
               jax.ShapeDtypeStruct((N, NE), jnp.float32)],
)

_tc3 = pl.pallas_call(
    _tc3_body, grid=(GRID,),
    in_specs=[_blk_spec(4), _deg_spec(),
              pl.BlockSpec((RB, NE), lambda i: (i, 0)),
              _full((NE, HID, HID)), _full((NE, HID)), _full((HID, HID))],
    out_specs=_blk_spec(4),
    out_shape=jax.ShapeDtypeStruct((4, N, 128), jnp.float32),
)

_tc4 = pl.pallas_call(
    _tc4_body, grid=(GRID,),
    in_specs=[_blk_spec(4), _deg_spec(), _full((1, HID)),
              _full((HID, OUT_DIM))],
    out_specs=_blk_spec(2),
    out_shape=jax.ShapeDtypeStruct((2, N, 128), jnp.float32),
)

_tc5 = pl.pallas_call(
    _tc5_body, grid=(GRID,),
    in_specs=[_blk_spec(2), _deg_spec(), _full((1, OUT_DIM))],
    out_specs=pl.BlockSpec((RB, OUT_DIM), lambda i: (i, 0)),
    out_shape=jax.ShapeDtypeStruct((N, OUT_DIM), jnp.float32),
)


def kernel(x, edge_index, W1, b1, w_gate, We, be, W2, b2, W3, b3):
    dst = edge_index[0]
    src = edge_index[1]
    pad = EP - E
    srcp = jnp.concatenate([src, jnp.zeros((pad,), jnp.int32)])
    dstp = jnp.concatenate([dst, jnp.full((pad,), N, jnp.int32)])
    ones128 = jnp.ones((EB, 128), jnp.float32)
    z128 = jnp.zeros((EB, 128), jnp.float32)
    zrows = jnp.zeros((RPT, 128), jnp.float32)
    srcoff = _srcoff(srcp.reshape(EP // EB, EB)).reshape(4, EP // GB, GB)
    dst2 = dstp.reshape(EP // GB, GB)

    _deg = _make_deg_sc()
    _segsum4 = _make_segsum(4)
    _segsum2 = _make_segsum(2)

    degraw = _deg(dstp, ones128, z128)

    y1 = _tc1(x, W1, degraw)
    s1 = _segsum4(y1.reshape(4 * N, 128), srcoff, dst2, zrows)
    y2, gates = _tc2(s1[:, :N], degraw, b1.reshape(1, -1), w_gate)
    s2 = _segsum4(y2.reshape(4 * N, 128), srcoff, dst2, zrows)
    y3 = _tc3(s2[:, :N], degraw, gates, We, be, W2)
    s3 = _segsum4(y3.reshape(4 * N, 128), srcoff, dst2, zrows)
    y4 = _tc4(s3[:, :N], degraw, b2.reshape(1, -1), W3)
    s4 = _segsum2(y4.reshape(2 * N, 128), srcoff, dst2, zrows)
    out = _tc5(s4[:, :N], degraw, b3.reshape(1, -1))
    return out
